# baseline (device time: 317350 ns/iter reference)
import jax
import jax.numpy as jnp
from jax import lax
from jax.experimental import pallas as pl
from jax.experimental.pallas import tpu as pltpu

N_DEV = 4
S_CHUNK = 512
N_SUB = 2
SUB_ROWS = S_CHUNK // N_SUB
N_TILES = 8


def kernel(O, Wo):
    B, S, H, D = O.shape
    K = H * D
    N = Wo.shape[1]
    NT = N // N_TILES

    O2 = O.astype(jnp.bfloat16)
    Wo2 = Wo.astype(jnp.bfloat16)

    def body(o_ref, w_ref, out_ref, send_buf, recv_buf, out_stage, o_stage,
             send_sems, recv_sems, copy_sems, o_sems, credit_sems):
        my = lax.axis_index("i")
        left = lax.rem(my + N_DEV - 1, N_DEV)
        right = lax.rem(my + 1, N_DEV)

        def rdma_for(nt, t):
            return pltpu.make_async_remote_copy(
                src_ref=send_buf.at[nt],
                dst_ref=recv_buf.at[nt, t % 2],
                send_sem=send_sems.at[nt],
                recv_sem=recv_sems.at[nt, t % 2],
                device_id=(right,),
                device_id_type=pl.DeviceIdType.MESH,
            )

        def dot_tile(o_c, nt):
            return lax.dot_general(
                o_c, w_ref[:, nt * NT:(nt + 1) * NT],
                dimension_numbers=(((2,), (0,)), ((), ())),
                preferred_element_type=jnp.float32,
            )

        def o_dma(c, row0, slot):
            return pltpu.make_async_copy(
                o_ref.at[:, pl.ds(c * S_CHUNK + row0, SUB_ROWS), :, :],
                o_stage.at[slot],
                o_sems.at[slot],
            )

        O_SCHED = []

        def load_o(k):
            c, row0 = O_SCHED[k]
            o_dma(c, row0, k % 2).wait()
            o4 = o_stage[k % 2]
            if k + 2 < len(O_SCHED):
                c2, row2 = O_SCHED[k + 2]
                o_dma(c2, row2, k % 2).start()
            return o4.reshape(B, SUB_ROWS, K)

        def signal_credit(nt):
            pl.semaphore_signal(
                credit_sems.at[nt], inc=1,
                device_id=(left,), device_id_type=pl.DeviceIdType.MESH,
            )

        store_count = [0]

        def out_copy(row0, nt, tot):
            k = store_count[0]
            store_count[0] += 1
            slot = k % 2
            dst = out_ref.at[:, pl.ds(row0, SUB_ROWS),
                             pl.ds(nt * NT, NT)]
            cp = pltpu.make_async_copy(out_stage.at[slot], dst,
                                       copy_sems.at[slot])
            if k >= 2:
                cp.wait()
            out_stage[slot] = tot
            cp.start()

        def seed(sub, t, k):
            o_c = load_o(k)
            for nt in range(N_TILES):
                p = dot_tile(o_c, nt)
                if t >= 2:
                    pl.semaphore_wait(credit_sems.at[nt], 1)
                if t >= 1:
                    rdma_for(nt, t - 1).wait_send()
                send_buf[nt] = p.astype(jnp.bfloat16)
                rdma_for(nt, t).start()

        def hop(sub, s, t, k, forward):
            row0 = sub * SUB_ROWS
            o_c = load_o(k)
            for nt in range(N_TILES):
                p = dot_tile(o_c, nt)
                rdma_for(nt, t).wait_recv()
                tot = p + recv_buf[nt, t % 2].astype(jnp.float32)
                if t + 2 <= 2 * (N_DEV - 1) - 1:
                    signal_credit(nt)
                if forward:
                    if t + 1 >= 2:
                        pl.semaphore_wait(credit_sems.at[nt], 1)
                    rdma_for(nt, t).wait_send()
                    send_buf[nt] = tot.astype(jnp.bfloat16)
                    rdma_for(nt, t + 1).start()
                else:
                    out_copy(row0, nt, tot)

        c0 = lax.rem(my + N_DEV - 1, N_DEV)
        cs = [lax.rem(my + 2 * N_DEV - s - 2, N_DEV) for s in range(3)]
        O_SCHED.extend([
            (c0, 0), (cs[0], 0), (cs[1], 0),
            (c0, SUB_ROWS), (cs[2], 0),
            (cs[0], SUB_ROWS), (cs[1], SUB_ROWS), (cs[2], SUB_ROWS),
        ])
        for k in range(2):
            o_dma(*O_SCHED[k], k).start()

        barrier_sem = pltpu.get_barrier_semaphore()
        for nbr in [left, right]:
            pl.semaphore_signal(
                barrier_sem, inc=1,
                device_id=(nbr,), device_id_type=pl.DeviceIdType.MESH,
            )
        pl.semaphore_wait(barrier_sem, 2)

        seed(0, 0, k=0)
        hop(0, 0, 0, k=1, forward=True)
        hop(0, 1, 1, k=2, forward=True)
        seed(1, 3, k=3)
        hop(0, 2, 2, k=4, forward=False)
        hop(1, 0, 3, k=5, forward=True)
        hop(1, 1, 4, k=6, forward=True)
        hop(1, 2, 5, k=7, forward=False)

        for nt in range(N_TILES):
            rdma_for(nt, 5).wait_send()
        for slot in range(2):
            pltpu.make_async_copy(
                out_stage.at[slot],
                out_ref.at[:, pl.ds(0, SUB_ROWS), pl.ds(slot * NT, NT)],
                copy_sems.at[slot],
            ).wait()

    return pl.pallas_call(
        body,
        out_shape=jax.ShapeDtypeStruct((B, S_CHUNK, N), jnp.float32),
        in_specs=[
            pl.BlockSpec(memory_space=pl.ANY),
            pl.BlockSpec(memory_space=pltpu.VMEM),
        ],
        out_specs=pl.BlockSpec(memory_space=pl.ANY),
        scratch_shapes=[
            pltpu.VMEM((N_TILES, B, SUB_ROWS, NT), jnp.bfloat16),
            pltpu.VMEM((N_TILES, 2, B, SUB_ROWS, NT), jnp.bfloat16),
            pltpu.VMEM((2, B, SUB_ROWS, NT), jnp.float32),
            pltpu.VMEM((2, B, SUB_ROWS, H, D), jnp.bfloat16),
            pltpu.SemaphoreType.DMA((N_TILES,)),
            pltpu.SemaphoreType.DMA((N_TILES, 2)),
            pltpu.SemaphoreType.DMA((2,)),
            pltpu.SemaphoreType.DMA((2,)),
            pltpu.SemaphoreType.REGULAR((N_TILES,)),
        ],
        compiler_params=pltpu.CompilerParams(collective_id=0),
    )(O2, Wo2)


# device time: 309205 ns/iter; 1.0263x vs baseline; 1.0263x over previous
import jax
import jax.numpy as jnp
from jax import lax
from jax.experimental import pallas as pl
from jax.experimental.pallas import tpu as pltpu

N_DEV = 4
S_CHUNK = 512
N_SUB = 2
SUB_ROWS = S_CHUNK // N_SUB
N_TILES = 8


def kernel(O, Wo):
    B, S, H, D = O.shape
    K = H * D
    N = Wo.shape[1]
    NT = N // N_TILES

    O2 = O.astype(jnp.bfloat16).reshape(B, S, K)
    Wo2 = Wo.astype(jnp.bfloat16)

    def body(o_ref, w_ref, out_ref, send_buf, recv_buf, out_stage, o_stage,
             send_sems, recv_sems, copy_sems, o_sems, credit_sems):
        my = lax.axis_index("i")
        left = lax.rem(my + N_DEV - 1, N_DEV)
        right = lax.rem(my + 1, N_DEV)

        def rdma_for(nt, t):
            return pltpu.make_async_remote_copy(
                src_ref=send_buf.at[nt],
                dst_ref=recv_buf.at[nt, t % 2],
                send_sem=send_sems.at[nt],
                recv_sem=recv_sems.at[nt, t % 2],
                device_id=(right,),
                device_id_type=pl.DeviceIdType.MESH,
            )

        def dot_tile(o_c, nt):
            return lax.dot_general(
                o_c, w_ref[:, nt * NT:(nt + 1) * NT],
                dimension_numbers=(((2,), (0,)), ((), ())),
                preferred_element_type=jnp.float32,
            )

        def o_dma(c, row0, slot):
            return pltpu.make_async_copy(
                o_ref.at[:, pl.ds(c * S_CHUNK + row0, SUB_ROWS), :],
                o_stage.at[slot],
                o_sems.at[slot],
            )

        O_SCHED = []

        def load_o(k):
            c, row0 = O_SCHED[k]
            o_dma(c, row0, k % 2).wait()
            o_c = o_stage[k % 2]
            if k + 2 < len(O_SCHED):
                c2, row2 = O_SCHED[k + 2]
                o_dma(c2, row2, k % 2).start()
            return o_c

        def signal_credit(nt):
            pl.semaphore_signal(
                credit_sems.at[nt], inc=1,
                device_id=(left,), device_id_type=pl.DeviceIdType.MESH,
            )

        store_count = [0]

        def out_copy(row0, nt, tot):
            k = store_count[0]
            store_count[0] += 1
            slot = k % 2
            dst = out_ref.at[:, pl.ds(row0, SUB_ROWS),
                             pl.ds(nt * NT, NT)]
            cp = pltpu.make_async_copy(out_stage.at[slot], dst,
                                       copy_sems.at[slot])
            if k >= 2:
                cp.wait()
            out_stage[slot] = tot
            cp.start()

        def seed(sub, t, k):
            o_c = load_o(k)
            for nt in range(N_TILES):
                p = dot_tile(o_c, nt)
                if t >= 2:
                    pl.semaphore_wait(credit_sems.at[nt], 1)
                if t >= 1:
                    rdma_for(nt, t - 1).wait_send()
                send_buf[nt] = p.astype(jnp.bfloat16)
                rdma_for(nt, t).start()

        def hop(sub, s, t, k, forward):
            row0 = sub * SUB_ROWS
            o_c = load_o(k)
            for nt in range(N_TILES):
                p = dot_tile(o_c, nt)
                rdma_for(nt, t).wait_recv()
                tot = p + recv_buf[nt, t % 2].astype(jnp.float32)
                if t + 2 <= 2 * (N_DEV - 1) - 1:
                    signal_credit(nt)
                if forward:
                    if t + 1 >= 2:
                        pl.semaphore_wait(credit_sems.at[nt], 1)
                    rdma_for(nt, t).wait_send()
                    send_buf[nt] = tot.astype(jnp.bfloat16)
                    rdma_for(nt, t + 1).start()
                else:
                    out_copy(row0, nt, tot)

        c0 = lax.rem(my + N_DEV - 1, N_DEV)
        cs = [lax.rem(my + 2 * N_DEV - s - 2, N_DEV) for s in range(3)]
        O_SCHED.extend([
            (c0, 0), (cs[0], 0), (cs[1], 0),
            (c0, SUB_ROWS), (cs[2], 0),
            (cs[0], SUB_ROWS), (cs[1], SUB_ROWS), (cs[2], SUB_ROWS),
        ])
        for k in range(2):
            o_dma(*O_SCHED[k], k).start()

        barrier_sem = pltpu.get_barrier_semaphore()
        for nbr in [left, right]:
            pl.semaphore_signal(
                barrier_sem, inc=1,
                device_id=(nbr,), device_id_type=pl.DeviceIdType.MESH,
            )
        pl.semaphore_wait(barrier_sem, 2)

        seed(0, 0, k=0)
        hop(0, 0, 0, k=1, forward=True)
        hop(0, 1, 1, k=2, forward=True)
        seed(1, 3, k=3)
        hop(0, 2, 2, k=4, forward=False)
        hop(1, 0, 3, k=5, forward=True)
        hop(1, 1, 4, k=6, forward=True)
        hop(1, 2, 5, k=7, forward=False)

        for nt in range(N_TILES):
            rdma_for(nt, 5).wait_send()
        for slot in range(2):
            pltpu.make_async_copy(
                out_stage.at[slot],
                out_ref.at[:, pl.ds(0, SUB_ROWS), pl.ds(slot * NT, NT)],
                copy_sems.at[slot],
            ).wait()

    return pl.pallas_call(
        body,
        out_shape=jax.ShapeDtypeStruct((B, S_CHUNK, N), jnp.float32),
        in_specs=[
            pl.BlockSpec(memory_space=pl.ANY),
            pl.BlockSpec(memory_space=pltpu.VMEM),
        ],
        out_specs=pl.BlockSpec(memory_space=pl.ANY),
        scratch_shapes=[
            pltpu.VMEM((N_TILES, B, SUB_ROWS, NT), jnp.bfloat16),
            pltpu.VMEM((N_TILES, 2, B, SUB_ROWS, NT), jnp.bfloat16),
            pltpu.VMEM((2, B, SUB_ROWS, NT), jnp.float32),
            pltpu.VMEM((2, B, SUB_ROWS, K), jnp.bfloat16),
            pltpu.SemaphoreType.DMA((N_TILES,)),
            pltpu.SemaphoreType.DMA((N_TILES, 2)),
            pltpu.SemaphoreType.DMA((2,)),
            pltpu.SemaphoreType.DMA((2,)),
            pltpu.SemaphoreType.REGULAR((N_TILES,)),
        ],
        compiler_params=pltpu.CompilerParams(collective_id=0),
    )(O2, Wo2)
